# initial kernel scaffold (unmeasured)
import jax
import jax.numpy as jnp
from jax import lax
from jax.experimental import pallas as pl
from jax.experimental.pallas import tpu as pltpu

N_DEV = 4
M_PER = 1024
K = 4096
N_PER = 2048
KT = 512
NKT = K // KT


def kernel(x, w_mat):
    def body(x_hbm, w_hbm, out_hbm, x_bf, xtmp, wbuf, wbf, stage,
             xsem, wsem, send_sems, recv_sems, out_sem):
        p = lax.axis_index("i")

        def x_dma(k):
            return pltpu.make_async_copy(
                x_hbm.at[:, pl.ds(k * KT, KT)], xtmp.at[k % 2], xsem.at[k % 2])

        pending = x_dma(0)
        pending.start()
        for k in range(NKT):
            nxt = None
            if k + 1 < NKT:
                nxt = x_dma(k + 1)
                nxt.start()
            pending.wait()
            x_bf[:, pl.ds(k * KT, KT)] = xtmp[k % 2].astype(jnp.bfloat16)
            pending = nxt

        targets = [1, 2, 3, 0]
        steps = []
        for oi, o in enumerate(targets):
            q = (p + o) % N_DEV
            for k in range(NKT):
                steps.append((oi, o, q, k))

        wdmas = [None] * len(steps)

        def start_w(t):
            _, _, q, k = steps[t]
            d = pltpu.make_async_copy(
                w_hbm.at[pl.ds(k * KT, KT), pl.ds(q * N_PER, N_PER)],
                wbuf.at[t % 2], wsem.at[t % 2])
            d.start()
            wdmas[t] = d

        start_w(0)
        send_rdmas = {}
        local_copy = None
        for t, (oi, o, q, k) in enumerate(steps):
            if t + 1 < len(steps):
                start_w(t + 1)
            s = oi % 2
            if k == 0 and oi >= 2:
                send_rdmas[targets[oi - 2]].wait_send()
            wdmas[t].wait()
            wbf[...] = wbuf[t % 2].astype(jnp.bfloat16)
            acc = jnp.dot(x_bf[:, k * KT:(k + 1) * KT], wbf[...],
                          preferred_element_type=jnp.float32)
            if k == 0:
                stage[s] = acc
            else:
                stage[s] = stage[s] + acc

            if k == NKT - 1:
                y = stage[s]
                stage[s] = y * jax.nn.sigmoid(y)
                if o == 0:
                    local_copy = pltpu.make_async_copy(
                        stage.at[s],
                        out_hbm.at[pl.ds(p * M_PER, M_PER), :],
                        out_sem)
                    local_copy.start()
                else:
                    rdma = pltpu.make_async_remote_copy(
                        src_ref=stage.at[s],
                        dst_ref=out_hbm.at[pl.ds(p * M_PER, M_PER), :],
                        send_sem=send_sems.at[s],
                        recv_sem=recv_sems.at[o - 1],
                        device_id=(q,),
                        device_id_type=pl.DeviceIdType.MESH)
                    rdma.start()
                    send_rdmas[o] = rdma

        send_rdmas[targets[-2]].wait_send()
        local_copy.wait()
        for o in (1, 2, 3):
            src = (p - o) % N_DEV
            recv = pltpu.make_async_remote_copy(
                src_ref=stage.at[0],
                dst_ref=out_hbm.at[pl.ds(src * M_PER, M_PER), :],
                send_sem=send_sems.at[0],
                recv_sem=recv_sems.at[o - 1],
                device_id=(p,),
                device_id_type=pl.DeviceIdType.MESH)
            recv.wait_recv()

    out_shape = jax.ShapeDtypeStruct((N_DEV * M_PER, N_PER), jnp.float32)
    return pl.pallas_call(
        body,
        out_shape=out_shape,
        in_specs=[
            pl.BlockSpec(memory_space=pltpu.ANY),
            pl.BlockSpec(memory_space=pltpu.ANY),
        ],
        out_specs=pl.BlockSpec(memory_space=pltpu.ANY),
        scratch_shapes=[
            pltpu.VMEM((M_PER, K), jnp.bfloat16),
            pltpu.VMEM((2, M_PER, KT), jnp.float32),
            pltpu.VMEM((2, KT, N_PER), jnp.float32),
            pltpu.VMEM((KT, N_PER), jnp.bfloat16),
            pltpu.VMEM((2, M_PER, N_PER), jnp.float32),
            pltpu.SemaphoreType.DMA((2,)),
            pltpu.SemaphoreType.DMA((2,)),
            pltpu.SemaphoreType.DMA((2,)),
            pltpu.SemaphoreType.DMA((3,)),
            pltpu.SemaphoreType.DMA,
        ],
        compiler_params=pltpu.CompilerParams(collective_id=0),
    )(x, w_mat)


# baseline (device time: 274281 ns/iter reference)
import jax
import jax.numpy as jnp
from jax import lax
from jax.experimental import pallas as pl
from jax.experimental.pallas import tpu as pltpu

N_DEV = 4
M_PER = 1024
K = 4096
N_PER = 2048
KT = 512
NKT = K // KT


def kernel(x, w_mat):
    def body(x_hbm, w_hbm, out_hbm, x_bf, xtmp, wbuf, wbf, stage,
             xsem, wsem, send_sems, recv_sems, out_sem):
        p = lax.axis_index("i")

        barrier_sem = pltpu.get_barrier_semaphore()
        for o in (1, 2, 3):
            pl.semaphore_signal(
                barrier_sem, inc=1,
                device_id=((p + o) % N_DEV,),
                device_id_type=pl.DeviceIdType.MESH)
        pl.semaphore_wait(barrier_sem, 3)

        def x_dma(k):
            return pltpu.make_async_copy(
                x_hbm.at[:, pl.ds(k * KT, KT)], xtmp.at[k % 2], xsem.at[k % 2])

        pending = x_dma(0)
        pending.start()
        for k in range(NKT):
            nxt = None
            if k + 1 < NKT:
                nxt = x_dma(k + 1)
                nxt.start()
            pending.wait()
            x_bf[:, pl.ds(k * KT, KT)] = xtmp[k % 2].astype(jnp.bfloat16)
            pending = nxt

        targets = [1, 2, 3, 0]
        steps = []
        for oi, o in enumerate(targets):
            q = (p + o) % N_DEV
            for k in range(NKT):
                steps.append((oi, o, q, k))

        wdmas = [None] * len(steps)

        def start_w(t):
            _, _, q, k = steps[t]
            d = pltpu.make_async_copy(
                w_hbm.at[pl.ds(k * KT, KT), pl.ds(q * N_PER, N_PER)],
                wbuf.at[t % 2], wsem.at[t % 2])
            d.start()
            wdmas[t] = d

        start_w(0)
        send_rdmas = {}
        local_copy = None
        for t, (oi, o, q, k) in enumerate(steps):
            if t + 1 < len(steps):
                start_w(t + 1)
            s = oi % 2
            if k == 0 and oi >= 2:
                send_rdmas[targets[oi - 2]].wait_send()
            wdmas[t].wait()
            wbf[...] = wbuf[t % 2].astype(jnp.bfloat16)
            acc = jnp.dot(x_bf[:, k * KT:(k + 1) * KT], wbf[...],
                          preferred_element_type=jnp.float32)
            if k == 0:
                stage[s] = acc
            else:
                stage[s] = stage[s] + acc

            if k == NKT - 1:
                y = stage[s]
                stage[s] = y * jax.nn.sigmoid(y)
                if o == 0:
                    local_copy = pltpu.make_async_copy(
                        stage.at[s],
                        out_hbm.at[pl.ds(p * M_PER, M_PER), :],
                        out_sem)
                    local_copy.start()
                else:
                    rdma = pltpu.make_async_remote_copy(
                        src_ref=stage.at[s],
                        dst_ref=out_hbm.at[pl.ds(p * M_PER, M_PER), :],
                        send_sem=send_sems.at[s],
                        recv_sem=recv_sems.at[o - 1],
                        device_id=(q,),
                        device_id_type=pl.DeviceIdType.MESH)
                    rdma.start()
                    send_rdmas[o] = rdma

        send_rdmas[targets[-2]].wait_send()
        local_copy.wait()
        for o in (1, 2, 3):
            src = (p - o) % N_DEV
            recv = pltpu.make_async_remote_copy(
                src_ref=stage.at[0],
                dst_ref=out_hbm.at[pl.ds(src * M_PER, M_PER), :],
                send_sem=send_sems.at[0],
                recv_sem=recv_sems.at[o - 1],
                device_id=(p,),
                device_id_type=pl.DeviceIdType.MESH)
            recv.wait_recv()

    out_shape = jax.ShapeDtypeStruct((N_DEV * M_PER, N_PER), jnp.float32)
    return pl.pallas_call(
        body,
        out_shape=out_shape,
        in_specs=[
            pl.BlockSpec(memory_space=pl.ANY),
            pl.BlockSpec(memory_space=pl.ANY),
        ],
        out_specs=pl.BlockSpec(memory_space=pl.ANY),
        scratch_shapes=[
            pltpu.VMEM((M_PER, K), jnp.bfloat16),
            pltpu.VMEM((2, M_PER, KT), jnp.float32),
            pltpu.VMEM((2, KT, N_PER), jnp.float32),
            pltpu.VMEM((KT, N_PER), jnp.bfloat16),
            pltpu.VMEM((2, M_PER, N_PER), jnp.float32),
            pltpu.SemaphoreType.DMA((2,)),
            pltpu.SemaphoreType.DMA((2,)),
            pltpu.SemaphoreType.DMA((2,)),
            pltpu.SemaphoreType.DMA((3,)),
            pltpu.SemaphoreType.DMA,
        ],
        compiler_params=pltpu.CompilerParams(
            collective_id=0, vmem_limit_bytes=60 * 1024 * 1024),
    )(x, w_mat)


# device time: 180516 ns/iter; 1.5194x vs baseline; 1.5194x over previous
import jax
import jax.numpy as jnp
from jax import lax
from jax.experimental import pallas as pl
from jax.experimental.pallas import tpu as pltpu

N_DEV = 4
M_PER = 1024
K = 4096
N_PER = 2048
KT = 512
NKT = K // KT


def kernel(x, w_mat):
    def body(x_hbm, w_hbm, out_hbm, x_bf, xtmp, wbuf, wbf, stage,
             send_bf, recv_bf, xsem, wsem, send_sems, recv_sems, out_sem):
        p = lax.axis_index("i")

        barrier_sem = pltpu.get_barrier_semaphore()
        for o in (1, 2, 3):
            pl.semaphore_signal(
                barrier_sem, inc=1,
                device_id=((p + o) % N_DEV,),
                device_id_type=pl.DeviceIdType.MESH)
        pl.semaphore_wait(barrier_sem, 3)

        targets = [1, 3, 2, 0]
        steps = []
        for oi, o in enumerate(targets):
            q = (p + o) % N_DEV
            for k in range(NKT):
                steps.append((oi, o, q, k))

        wdmas = [None] * len(steps)

        def start_w(t):
            _, _, q, k = steps[t]
            d = pltpu.make_async_copy(
                w_hbm.at[pl.ds(k * KT, KT), pl.ds(q * N_PER, N_PER)],
                wbuf.at[t % 2], wsem.at[t % 2])
            d.start()
            wdmas[t] = d

        start_w(0)

        def x_dma(k):
            return pltpu.make_async_copy(
                x_hbm.at[:, pl.ds(k * KT, KT)], xtmp.at[k % 2], xsem.at[k % 2])

        pending = x_dma(0)
        pending.start()
        for k in range(NKT):
            nxt = None
            if k + 1 < NKT:
                nxt = x_dma(k + 1)
                nxt.start()
            pending.wait()
            x_bf[:, pl.ds(k * KT, KT)] = xtmp[k % 2].astype(jnp.bfloat16)
            pending = nxt

        send_rdmas = {}
        local_copy = None
        for t, (oi, o, q, k) in enumerate(steps):
            if t + 1 < len(steps):
                start_w(t + 1)
            if k == 0 and oi == 2:
                send_rdmas[targets[0]].wait_send()
            wdmas[t].wait()
            wbf[...] = wbuf[t % 2].astype(jnp.bfloat16)
            acc = jnp.dot(x_bf[:, k * KT:(k + 1) * KT], wbf[...],
                          preferred_element_type=jnp.float32)
            if k == 0:
                stage[...] = acc
            else:
                stage[...] = stage[...] + acc

            if k == NKT - 1:
                y = stage[...]
                if o == 0:
                    stage[...] = y * jax.nn.sigmoid(y)
                    local_copy = pltpu.make_async_copy(
                        stage, out_hbm.at[pl.ds(p * M_PER, M_PER), :],
                        out_sem)
                    local_copy.start()
                else:
                    s = oi % 2
                    send_bf[s] = (y * jax.nn.sigmoid(y)).astype(jnp.bfloat16)
                    rdma = pltpu.make_async_remote_copy(
                        src_ref=send_bf.at[s],
                        dst_ref=recv_bf.at[o - 1],
                        send_sem=send_sems.at[s],
                        recv_sem=recv_sems.at[o - 1],
                        device_id=(q,),
                        device_id_type=pl.DeviceIdType.MESH)
                    rdma.start()
                    send_rdmas[o] = rdma

        send_rdmas[targets[1]].wait_send()
        send_rdmas[targets[2]].wait_send()
        local_copy.wait()
        for o in (1, 3, 2):
            src = (p - o) % N_DEV
            recv = pltpu.make_async_remote_copy(
                src_ref=send_bf.at[0],
                dst_ref=recv_bf.at[o - 1],
                send_sem=send_sems.at[0],
                recv_sem=recv_sems.at[o - 1],
                device_id=(p,),
                device_id_type=pl.DeviceIdType.MESH)
            recv.wait_recv()
            stage[...] = recv_bf[o - 1].astype(jnp.float32)
            out_copy = pltpu.make_async_copy(
                stage, out_hbm.at[pl.ds(src * M_PER, M_PER), :], out_sem)
            out_copy.start()
            out_copy.wait()

    out_shape = jax.ShapeDtypeStruct((N_DEV * M_PER, N_PER), jnp.float32)
    return pl.pallas_call(
        body,
        out_shape=out_shape,
        in_specs=[
            pl.BlockSpec(memory_space=pl.ANY),
            pl.BlockSpec(memory_space=pl.ANY),
        ],
        out_specs=pl.BlockSpec(memory_space=pl.ANY),
        scratch_shapes=[
            pltpu.VMEM((M_PER, K), jnp.bfloat16),
            pltpu.VMEM((2, M_PER, KT), jnp.float32),
            pltpu.VMEM((2, KT, N_PER), jnp.float32),
            pltpu.VMEM((KT, N_PER), jnp.bfloat16),
            pltpu.VMEM((M_PER, N_PER), jnp.float32),
            pltpu.VMEM((2, M_PER, N_PER), jnp.bfloat16),
            pltpu.VMEM((3, M_PER, N_PER), jnp.bfloat16),
            pltpu.SemaphoreType.DMA((2,)),
            pltpu.SemaphoreType.DMA((2,)),
            pltpu.SemaphoreType.DMA((2,)),
            pltpu.SemaphoreType.DMA((3,)),
            pltpu.SemaphoreType.DMA,
        ],
        compiler_params=pltpu.CompilerParams(
            collective_id=0, vmem_limit_bytes=60 * 1024 * 1024),
    )(x, w_mat)
